# baseline (speedup 1.0000x reference)
import jax
import jax.numpy as jnp
from jax.experimental import pallas as pl

BATCH = 16384
K = 1000
L = 16
BLK = 1024


def _body(uft_ref, ift_ref, uwt_ref, iwt_ref, ibt_ref, out_ref):
    uft = uft_ref[...]
    ift = ift_ref[...]
    ul = jnp.dot(uwt_ref[...], uft, preferred_element_type=jnp.float32)
    il = jnp.dot(iwt_ref[...], ift, preferred_element_type=jnp.float32)
    bias = jnp.dot(ibt_ref[...], ift, preferred_element_type=jnp.float32)
    out_ref[...] = jnp.sum(ul * il, axis=0) + bias[0]


def kernel(user_features, item_features, user_latent_w, item_latent_w, item_biases_w):
    uft = user_features.T
    ift = item_features.T
    uwt = user_latent_w.T
    iwt = item_latent_w.T
    ibt = item_biases_w.T
    grid = (BATCH // BLK,)
    return pl.pallas_call(
        _body,
        grid=grid,
        in_specs=[
            pl.BlockSpec((K, BLK), lambda i: (0, i)),
            pl.BlockSpec((K, BLK), lambda i: (0, i)),
            pl.BlockSpec((L, K), lambda i: (0, 0)),
            pl.BlockSpec((L, K), lambda i: (0, 0)),
            pl.BlockSpec((1, K), lambda i: (0, 0)),
        ],
        out_specs=pl.BlockSpec((BLK,), lambda i: (i,)),
        out_shape=jax.ShapeDtypeStruct((BATCH,), jnp.float32),
    )(uft, ift, uwt, iwt, ibt)
